# Initial kernel scaffold; baseline (speedup 1.0000x reference)
import jax
import jax.numpy as jnp
from jax.experimental import pallas as pl


def kernel(samples, bin_edges):
    raise NotImplementedError("write your pallas kernel here")


# speedup vs baseline: 955.8489x; 955.8489x over previous
import jax
import jax.numpy as jnp
from jax.experimental import pallas as pl
from jax.experimental.pallas import tpu as pltpu

_B, _C, _L, _NBINS = 256, 32, 128, 32
_NB = 4


def _tc_body(x_ref, e_ref, a_ref, lo_ref, hi_ref, o_ref):
    x = x_ref[...][:, :, None, :]
    e = e_ref[...][None, :, :, :]
    a = a_ref[...][None, :, :, :]
    lo = lo_ref[...][None, :, :, :]
    hi = hi_ref[...][None, :, :, :]
    r = (x - e) * a
    o_ref[...] = jnp.minimum(jnp.maximum(r, lo), hi)


def kernel(samples, bin_edges):
    B, C, L = samples.shape
    nb = bin_edges.shape[1] - 1
    e = bin_edges[:, :-1]
    a = 1.0 / (bin_edges[:, 1:] - bin_edges[:, :-1])
    jvec = jnp.arange(nb, dtype=jnp.float32)
    lo = jnp.where(jvec == 0, -jnp.inf, 0.0)[None, :] * jnp.ones((C, 1), jnp.float32)
    hi = jnp.where(jvec == nb - 1, jnp.inf, 1.0)[None, :] * jnp.ones((C, 1), jnp.float32)
    e, a, lo, hi = (t[..., None] for t in (e, a, lo, hi))

    grid = (B // _NB,)
    tab_spec = pl.BlockSpec((C, nb, 1), lambda i: (0, 0, 0))
    out = pl.pallas_call(
        _tc_body,
        grid=grid,
        in_specs=[
            pl.BlockSpec((_NB, C, L), lambda i: (i, 0, 0)),
            tab_spec, tab_spec, tab_spec, tab_spec,
        ],
        out_specs=pl.BlockSpec((_NB, C, nb, L), lambda i: (i, 0, 0, 0)),
        out_shape=jax.ShapeDtypeStruct((B, C, nb, L), jnp.float32),
    )(samples, e, a, lo, hi)
    return out.reshape(B, C * nb, L)
